# Initial kernel scaffold; baseline (speedup 1.0000x reference)
import functools

import jax
import jax.numpy as jnp
from jax import lax
from jax.experimental import pallas as pl
from jax.experimental.pallas import tpu as pltpu
from jax.experimental.pallas import tpu_sc as plsc

N = 10000
E = 640000
F = 128
HID = 64

NSUB = 16
NPAD = 10240
NODES_PER_SUB = NPAD // NSUB
EDGES_PER_SUB = E // NSUB
CHUNK = 80
NCHUNKS = EDGES_PER_SUB // CHUNK
XROWS = 80


def _zero16():
    return jnp.zeros((16,), jnp.float32)


def _sc_body(x_hbm, row_hbm, col_hbm, w_hbm,
             po_hbm, pi_hbm, xso_hbm, xsi_hbm,
             P_sp, deg_sp,
             inv_v, xrow_v, sidx_v, didx_v, wch_v, rows_v, sem):
    c = lax.axis_index("c")
    s = lax.axis_index("s")
    base_n = s * NODES_PER_SUB

    def zrow(r, _):
        for j in range(F // 16):
            xrow_v[r, pl.ds(16 * j, 16)] = _zero16()
        return 0
    lax.fori_loop(0, XROWS, zrow, 0)

    def zinv(j, _):
        inv_v[pl.ds(16 * j, 16)] = _zero16()
        return 0
    lax.fori_loop(0, NODES_PER_SUB // 16, zinv, 0)

    for k in range(NODES_PER_SUB // XROWS):
        pltpu.sync_copy(xrow_v, P_sp.at[pl.ds(base_n + k * XROWS, XROWS), :])
    pltpu.sync_copy(inv_v, deg_sp.at[pl.ds(base_n, NODES_PER_SUB)])
    plsc.subcore_barrier()

    def direction(src_hbm, dst_hbm, p_out, xs_out):
        ebase = s * EDGES_PER_SUB

        def dega(j, _):
            off = ebase + j * CHUNK
            pltpu.sync_copy(src_hbm.at[pl.ds(off, CHUNK)], sidx_v)
            pltpu.sync_copy(w_hbm.at[pl.ds(off, CHUNK)], wch_v)
            pltpu.sync_copy(wch_v, deg_sp.at[sidx_v], add=True)
            return 0
        lax.fori_loop(0, NCHUNKS, dega, 0)
        plsc.subcore_barrier()

        pltpu.sync_copy(deg_sp.at[pl.ds(base_n, NODES_PER_SUB)], inv_v)

        def invb(j, _):
            v = inv_v[pl.ds(16 * j, 16)]
            safe = jnp.where(v > 0.0, v, 1.0)
            inv_v[pl.ds(16 * j, 16)] = jnp.where(v > 0.0, 1.0 / safe, 0.0)
            return 0
        lax.fori_loop(0, NODES_PER_SUB // 16, invb, 0)

        nch = jnp.maximum(0, jnp.minimum(NODES_PER_SUB, N - base_n)) // XROWS

        def xsb(k, _):
            noff = base_n + k * XROWS
            pltpu.sync_copy(x_hbm.at[pl.ds(noff, XROWS), :], xrow_v)

            def rowb(r, _):
                bidx = jnp.full((16,), k * XROWS + r, jnp.int32)
                bv = plsc.load_gather(inv_v, [bidx])
                for j in range(F // 16):
                    t = xrow_v[r, pl.ds(16 * j, 16)]
                    xrow_v[r, pl.ds(16 * j, 16)] = t * bv
                return 0
            lax.fori_loop(0, XROWS, rowb, 0)
            pltpu.sync_copy(xrow_v, xs_out.at[pl.ds(noff, XROWS), :])
            return 0
        lax.fori_loop(0, nch, xsb, 0)
        plsc.subcore_barrier()

        def edge(j, _):
            off = ebase + j * CHUNK
            pltpu.sync_copy(src_hbm.at[pl.ds(off, CHUNK)], sidx_v)
            pltpu.sync_copy(dst_hbm.at[pl.ds(off, CHUNK)], didx_v)
            pltpu.async_copy(xs_out.at[sidx_v], rows_v, sem).wait()
            pltpu.sync_copy(rows_v, P_sp.at[didx_v], add=True)
            return 0
        lax.fori_loop(0, NCHUNKS, edge, 0)
        plsc.subcore_barrier()

        pltpu.sync_copy(P_sp.at[pl.ds(base_n, NODES_PER_SUB), :],
                        p_out.at[pl.ds(base_n, NODES_PER_SUB), :])

    @pl.when(c == 0)
    def _():
        direction(row_hbm, col_hbm, po_hbm, xso_hbm)

    @pl.when(c == 1)
    def _():
        direction(col_hbm, row_hbm, pi_hbm, xsi_hbm)


def _sc_prop(x, row, col, w):
    mesh = plsc.VectorSubcoreMesh(core_axis_name="c", subcore_axis_name="s")
    f32 = jnp.float32
    kern = functools.partial(
        pl.kernel,
        mesh=mesh,
        out_type=[
            jax.ShapeDtypeStruct((NPAD, F), f32),
            jax.ShapeDtypeStruct((NPAD, F), f32),
            jax.ShapeDtypeStruct((NPAD, F), f32),
            jax.ShapeDtypeStruct((NPAD, F), f32),
        ],
        scratch_types=[
            pltpu.VMEM_SHARED((NPAD, F), f32),
            pltpu.VMEM_SHARED((NPAD,), f32),
            pltpu.VMEM((NODES_PER_SUB,), f32),
            pltpu.VMEM((XROWS, F), f32),
            pltpu.VMEM((CHUNK,), jnp.int32),
            pltpu.VMEM((CHUNK,), jnp.int32),
            pltpu.VMEM((CHUNK,), f32),
            pltpu.VMEM((CHUNK, F), f32),
            pltpu.SemaphoreType.DMA,
        ],
    )(_sc_body)
    return kern(x, row, col, w)


BN = 2000


def _tc_body(x_ref, po_ref, pi_ref, wz_ref, wh_ref, bz_ref, bh_ref,
             wl_ref, bl_ref, o_ref):
    cat = jnp.concatenate([x_ref[...], po_ref[...], pi_ref[...]], axis=1)
    z = jax.nn.sigmoid(
        jnp.dot(cat, wz_ref[...], preferred_element_type=jnp.float32)
        + bz_ref[...])
    ht = jnp.tanh(
        jnp.dot(cat, wh_ref[...], preferred_element_type=jnp.float32)
        + bh_ref[...])
    h = (1.0 - z) * ht
    o_ref[...] = (
        jnp.dot(jnp.maximum(h, 0.0), wl_ref[...],
                preferred_element_type=jnp.float32)
        + bl_ref[...])


def _tc_tail(x, p_o, p_i, wz, wh, bz, bh, wl, bl):
    grid = (N // BN,)

    def full(i):
        return (0, 0)

    return pl.pallas_call(
        _tc_body,
        grid=grid,
        in_specs=[
            pl.BlockSpec((BN, F), lambda i: (i, 0)),
            pl.BlockSpec((BN, F), lambda i: (i, 0)),
            pl.BlockSpec((BN, F), lambda i: (i, 0)),
            pl.BlockSpec((3 * F, HID), full),
            pl.BlockSpec((3 * F, HID), full),
            pl.BlockSpec((1, HID), full),
            pl.BlockSpec((1, HID), full),
            pl.BlockSpec((HID, 1), full),
            pl.BlockSpec((1, 1), full),
        ],
        out_specs=pl.BlockSpec((BN, 1), lambda i: (i, 0)),
        out_shape=jax.ShapeDtypeStruct((N, 1), jnp.float32),
    )(x, p_o, p_i, wz, wh, bz, bh, wl, bl)


def kernel(x, edge_index, edge_weight, W_z, b_z, W_r, b_r, W_h, b_h,
           W_lin, b_lin):
    del W_r, b_r
    ei = edge_index.astype(jnp.int32)
    row, col = ei[0], ei[1]
    p_o, p_i, _, _ = _sc_prop(x, row, col, edge_weight)

    def cat_w(W):
        return jnp.concatenate([W[0, 0, :F] + W[1, 0, :F],
                                W[0, 1, :F], W[1, 1, :F]], axis=0)

    out = _tc_tail(
        x, p_o, p_i, cat_w(W_z), cat_w(W_h),
        b_z.reshape(1, HID), b_h.reshape(1, HID),
        W_lin, b_lin.reshape(1, 1))
    return out


# speedup vs baseline: 16.4428x; 16.4428x over previous
import functools

import jax
import jax.numpy as jnp
from jax import lax
from jax.experimental import pallas as pl
from jax.experimental.pallas import tpu as pltpu
from jax.experimental.pallas import tpu_sc as plsc

N = 10000
E = 640000
F = 128
HID = 64

NSUB = 16
NPAD = 10240
NODES_PER_SUB = NPAD // NSUB
EDGES_PER_SUB = E // NSUB
CHUNK = 80
NCHUNKS = EDGES_PER_SUB // CHUNK


def _zero16():
    return jnp.zeros((16,), jnp.float32)


def _sc_deg_body(row_hbm, col_hbm, w_hbm, dego_hbm, degi_hbm,
                 deg_sp, zv_v, idx_v, wch_v):
    c = lax.axis_index("c")
    s = lax.axis_index("s")
    base_n = s * NODES_PER_SUB
    ebase = s * EDGES_PER_SUB

    def zinv(j, _):
        zv_v[pl.ds(16 * j, 16)] = _zero16()
        return 0
    lax.fori_loop(0, NODES_PER_SUB // 16, zinv, 0)
    pltpu.sync_copy(zv_v, deg_sp.at[pl.ds(base_n, NODES_PER_SUB)])
    plsc.subcore_barrier()

    def accum(src_hbm):
        def dega(j, _):
            off = ebase + j * CHUNK
            pltpu.sync_copy(src_hbm.at[pl.ds(off, CHUNK)], idx_v)
            pltpu.sync_copy(w_hbm.at[pl.ds(off, CHUNK)], wch_v)
            pltpu.sync_copy(wch_v, deg_sp.at[idx_v], add=True)
            return 0
        lax.fori_loop(0, NCHUNKS, dega, 0)

    @pl.when(c == 0)
    def _():
        accum(row_hbm)

    @pl.when(c == 1)
    def _():
        accum(col_hbm)

    plsc.subcore_barrier()

    @pl.when(c == 0)
    def _():
        pltpu.sync_copy(deg_sp.at[pl.ds(base_n, NODES_PER_SUB)],
                        dego_hbm.at[pl.ds(base_n, NODES_PER_SUB)])

    @pl.when(c == 1)
    def _():
        pltpu.sync_copy(deg_sp.at[pl.ds(base_n, NODES_PER_SUB)],
                        degi_hbm.at[pl.ds(base_n, NODES_PER_SUB)])


def _sc_deg(row, col, w):
    mesh = plsc.VectorSubcoreMesh(core_axis_name="c", subcore_axis_name="s")
    f32 = jnp.float32
    kern = functools.partial(
        pl.kernel,
        mesh=mesh,
        out_type=[
            jax.ShapeDtypeStruct((NPAD,), f32),
            jax.ShapeDtypeStruct((NPAD,), f32),
        ],
        scratch_types=[
            pltpu.VMEM_SHARED((NPAD,), f32),
            pltpu.VMEM((NODES_PER_SUB,), f32),
            pltpu.VMEM((CHUNK,), jnp.int32),
            pltpu.VMEM((CHUNK,), f32),
        ],
    )(_sc_deg_body)
    return kern(row, col, w)


def _sc_prop_body(xso_hbm, xsi_hbm, row_hbm, col_hbm, po_hbm, pi_hbm,
                  P_sp, zrow_v, sidx_v, didx_v, rows_v, sem):
    c = lax.axis_index("c")
    s = lax.axis_index("s")
    base_n = s * NODES_PER_SUB
    ebase = s * EDGES_PER_SUB

    def zrow(r, _):
        for j in range(F // 16):
            zrow_v[r, pl.ds(16 * j, 16)] = _zero16()
        return 0
    lax.fori_loop(0, CHUNK, zrow, 0)
    for k in range(NODES_PER_SUB // CHUNK):
        pltpu.sync_copy(zrow_v, P_sp.at[pl.ds(base_n + k * CHUNK, CHUNK), :])
    plsc.subcore_barrier()

    def edges(src_hbm, dst_hbm, xs_hbm):
        def edge(j, _):
            off = ebase + j * CHUNK
            pltpu.sync_copy(src_hbm.at[pl.ds(off, CHUNK)], sidx_v)
            pltpu.sync_copy(dst_hbm.at[pl.ds(off, CHUNK)], didx_v)
            pltpu.async_copy(xs_hbm.at[sidx_v], rows_v, sem).wait()
            pltpu.sync_copy(rows_v, P_sp.at[didx_v], add=True)
            return 0
        lax.fori_loop(0, NCHUNKS, edge, 0)

    @pl.when(c == 0)
    def _():
        edges(row_hbm, col_hbm, xso_hbm)

    @pl.when(c == 1)
    def _():
        edges(col_hbm, row_hbm, xsi_hbm)

    plsc.subcore_barrier()

    @pl.when(c == 0)
    def _():
        pltpu.sync_copy(P_sp.at[pl.ds(base_n, NODES_PER_SUB), :],
                        po_hbm.at[pl.ds(base_n, NODES_PER_SUB), :])

    @pl.when(c == 1)
    def _():
        pltpu.sync_copy(P_sp.at[pl.ds(base_n, NODES_PER_SUB), :],
                        pi_hbm.at[pl.ds(base_n, NODES_PER_SUB), :])


def _sc_prop(xs_o, xs_i, row, col):
    mesh = plsc.VectorSubcoreMesh(core_axis_name="c", subcore_axis_name="s")
    f32 = jnp.float32
    kern = functools.partial(
        pl.kernel,
        mesh=mesh,
        out_type=[
            jax.ShapeDtypeStruct((NPAD, F), f32),
            jax.ShapeDtypeStruct((NPAD, F), f32),
        ],
        scratch_types=[
            pltpu.VMEM_SHARED((NPAD, F), f32),
            pltpu.VMEM((CHUNK, F), f32),
            pltpu.VMEM((CHUNK,), jnp.int32),
            pltpu.VMEM((CHUNK,), jnp.int32),
            pltpu.VMEM((CHUNK, F), f32),
            pltpu.SemaphoreType.DMA,
        ],
    )(_sc_prop_body)
    return kern(xs_o, xs_i, row, col)


BN = 2000


def _tc_scale_body(x_ref, dego_ref, degi_ref, xso_ref, xsi_ref):
    x = x_ref[...]
    do = dego_ref[...]
    di = degi_ref[...]
    inv_o = jnp.where(do > 0.0, 1.0 / jnp.where(do > 0.0, do, 1.0), 0.0)
    inv_i = jnp.where(di > 0.0, 1.0 / jnp.where(di > 0.0, di, 1.0), 0.0)
    xso_ref[...] = x * inv_o
    xsi_ref[...] = x * inv_i


def _tc_scale(x, deg_o, deg_i):
    grid = (N // BN,)
    row_block = pl.BlockSpec((BN, F), lambda i: (i, 0))
    deg_block = pl.BlockSpec((BN, 1), lambda i: (i, 0))
    return pl.pallas_call(
        _tc_scale_body,
        grid=grid,
        in_specs=[row_block, deg_block, deg_block],
        out_specs=[row_block, row_block],
        out_shape=[
            jax.ShapeDtypeStruct((N, F), jnp.float32),
            jax.ShapeDtypeStruct((N, F), jnp.float32),
        ],
    )(x, deg_o, deg_i)


def _tc_tail_body(x_ref, po_ref, pi_ref, wz_ref, wh_ref, bz_ref, bh_ref,
                  wl_ref, bl_ref, o_ref):
    cat = jnp.concatenate([x_ref[...], po_ref[...], pi_ref[...]], axis=1)
    z = jax.nn.sigmoid(
        jnp.dot(cat, wz_ref[...], preferred_element_type=jnp.float32)
        + bz_ref[...])
    ht = jnp.tanh(
        jnp.dot(cat, wh_ref[...], preferred_element_type=jnp.float32)
        + bh_ref[...])
    h = (1.0 - z) * ht
    o_ref[...] = (
        jnp.dot(jnp.maximum(h, 0.0), wl_ref[...],
                preferred_element_type=jnp.float32)
        + bl_ref[...])


def _tc_tail(x, p_o, p_i, wz, wh, bz, bh, wl, bl):
    grid = (N // BN,)

    def full(i):
        return (0, 0)

    return pl.pallas_call(
        _tc_tail_body,
        grid=grid,
        in_specs=[
            pl.BlockSpec((BN, F), lambda i: (i, 0)),
            pl.BlockSpec((BN, F), lambda i: (i, 0)),
            pl.BlockSpec((BN, F), lambda i: (i, 0)),
            pl.BlockSpec((3 * F, HID), full),
            pl.BlockSpec((3 * F, HID), full),
            pl.BlockSpec((1, HID), full),
            pl.BlockSpec((1, HID), full),
            pl.BlockSpec((HID, 1), full),
            pl.BlockSpec((1, 1), full),
        ],
        out_specs=pl.BlockSpec((BN, 1), lambda i: (i, 0)),
        out_shape=jax.ShapeDtypeStruct((N, 1), jnp.float32),
    )(x, p_o, p_i, wz, wh, bz, bh, wl, bl)


def kernel(x, edge_index, edge_weight, W_z, b_z, W_r, b_r, W_h, b_h,
           W_lin, b_lin):
    del W_r, b_r
    ei = edge_index.astype(jnp.int32)
    row, col = ei[0], ei[1]

    deg_o, deg_i = _sc_deg(row, col, edge_weight)
    xs_o, xs_i = _tc_scale(
        x, deg_o[:N].reshape(N, 1), deg_i[:N].reshape(N, 1))
    p_o, p_i = _sc_prop(xs_o, xs_i, row, col)

    def cat_w(W):
        return jnp.concatenate([W[0, 0, :F] + W[1, 0, :F],
                                W[0, 1, :F], W[1, 1, :F]], axis=0)

    out = _tc_tail(
        x, p_o, p_i, cat_w(W_z), cat_w(W_h),
        b_z.reshape(1, HID), b_h.reshape(1, HID),
        W_lin, b_lin.reshape(1, 1))
    return out


# speedup vs baseline: 59.6897x; 3.6301x over previous
import functools

import jax
import jax.numpy as jnp
from jax import lax
from jax.experimental import pallas as pl
from jax.experimental.pallas import tpu as pltpu
from jax.experimental.pallas import tpu_sc as plsc

N = 10000
E = 640000
F = 128
HID = 64

NSUB = 16
NPAD = 10240
NODES_PER_SUB = NPAD // NSUB
EDGES_PER_SUB = E // NSUB
CHUNK = 80
NCHUNKS = EDGES_PER_SUB // CHUNK

NBUF = 4
NGROUPS = NCHUNKS // NBUF
DEG_K = 10
DEG_G = NCHUNKS // DEG_K


def _zero16():
    return jnp.zeros((16,), jnp.float32)


def _sc_deg_body(row_hbm, col_hbm, w_hbm, dego_hbm, degi_hbm,
                 deg_sp, zv_v, idxb_v, wb_v, lsem, ssem):
    c = lax.axis_index("c")
    s = lax.axis_index("s")
    base_n = s * NODES_PER_SUB
    ebase = s * EDGES_PER_SUB

    def zinv(j, _):
        zv_v[pl.ds(16 * j, 16)] = _zero16()
        return 0
    lax.fori_loop(0, NODES_PER_SUB // 16, zinv, 0)
    pltpu.sync_copy(zv_v, deg_sp.at[pl.ds(base_n, NODES_PER_SUB)])

    @pl.when(c == 0)
    def _():
        pltpu.async_copy(row_hbm.at[pl.ds(ebase, EDGES_PER_SUB)], idxb_v,
                         lsem)

    @pl.when(c == 1)
    def _():
        pltpu.async_copy(col_hbm.at[pl.ds(ebase, EDGES_PER_SUB)], idxb_v,
                         lsem)

    pltpu.async_copy(w_hbm.at[pl.ds(ebase, EDGES_PER_SUB)], wb_v, lsem)
    pltpu.make_async_copy(
        row_hbm.at[pl.ds(ebase, EDGES_PER_SUB)], idxb_v, lsem).wait()
    pltpu.make_async_copy(
        w_hbm.at[pl.ds(ebase, EDGES_PER_SUB)], wb_v, lsem).wait()

    plsc.subcore_barrier()

    def grp(g, _):
        for b in range(DEG_K):
            off = pl.multiple_of((g * DEG_K + b) * CHUNK, 8)
            pltpu.async_copy(wb_v.at[pl.ds(off, CHUNK)],
                             deg_sp.at[idxb_v.at[pl.ds(off, CHUNK)]], ssem,
                             add=True)
        for b in range(DEG_K):
            off = pl.multiple_of((g * DEG_K + b) * CHUNK, 8)
            pltpu.make_async_copy(wb_v.at[pl.ds(off, CHUNK)],
                                  deg_sp.at[idxb_v.at[pl.ds(off, CHUNK)]],
                                  ssem).wait()
        return 0
    lax.fori_loop(0, DEG_G, grp, 0)

    plsc.subcore_barrier()

    @pl.when(c == 0)
    def _():
        pltpu.sync_copy(deg_sp.at[pl.ds(base_n, NODES_PER_SUB)],
                        dego_hbm.at[pl.ds(base_n, NODES_PER_SUB)])

    @pl.when(c == 1)
    def _():
        pltpu.sync_copy(deg_sp.at[pl.ds(base_n, NODES_PER_SUB)],
                        degi_hbm.at[pl.ds(base_n, NODES_PER_SUB)])


def _sc_deg(row, col, w):
    mesh = plsc.VectorSubcoreMesh(core_axis_name="c", subcore_axis_name="s")
    f32 = jnp.float32
    kern = functools.partial(
        pl.kernel,
        mesh=mesh,
        out_type=[
            jax.ShapeDtypeStruct((NPAD,), f32),
            jax.ShapeDtypeStruct((NPAD,), f32),
        ],
        scratch_types=[
            pltpu.VMEM_SHARED((NPAD,), f32),
            pltpu.VMEM((NODES_PER_SUB,), f32),
            pltpu.VMEM((EDGES_PER_SUB,), jnp.int32),
            pltpu.VMEM((EDGES_PER_SUB,), f32),
            pltpu.SemaphoreType.DMA,
            pltpu.SemaphoreType.DMA,
        ],
    )(_sc_deg_body)
    return kern(row, col, w)


BLK = NBUF * CHUNK


def _sc_prop_body(xso_hbm, xsi_hbm, row_hbm, col_hbm, po_hbm, pi_hbm,
                  P_sp, sidx_v, didx_v, rows0, rows1, rows2, rows3,
                  lsem, g0, g1, g2, g3, s0, s1, s2, s3):
    c = lax.axis_index("c")
    s = lax.axis_index("s")
    base_n = s * NODES_PER_SUB
    ebase = s * EDGES_PER_SUB
    rows = (rows0, rows1, rows2, rows3)
    gsem = (g0, g1, g2, g3)
    ssem = (s0, s1, s2, s3)

    def zrow(r, _):
        for j in range(F // 16):
            rows0[r, pl.ds(16 * j, 16)] = _zero16()
        return 0
    lax.fori_loop(0, CHUNK, zrow, 0)
    for k in range(NODES_PER_SUB // CHUNK):
        pltpu.sync_copy(rows0, P_sp.at[pl.ds(base_n + k * CHUNK, CHUNK), :])
    plsc.subcore_barrier()

    def edges(src_hbm, dst_hbm, xs_hbm):
        pltpu.sync_copy(src_hbm.at[pl.ds(ebase, BLK)],
                        sidx_v.at[pl.ds(0, BLK)])
        pltpu.sync_copy(dst_hbm.at[pl.ds(ebase, BLK)],
                        didx_v.at[pl.ds(0, BLK)])
        for b in range(NBUF):
            o = b * CHUNK
            pltpu.async_copy(xs_hbm.at[sidx_v.at[pl.ds(o, CHUNK)]],
                             rows[b], gsem[b])

        def grp(g, _):
            p = lax.rem(g, 2) * BLK
            pn = pl.multiple_of(lax.rem(g + 1, 2) * BLK, 8)
            nxt = ebase + (g + 1) * BLK
            pltpu.async_copy(src_hbm.at[pl.ds(nxt, BLK)],
                             sidx_v.at[pl.ds(pn, BLK)], lsem)
            pltpu.async_copy(dst_hbm.at[pl.ds(nxt, BLK)],
                             didx_v.at[pl.ds(pn, BLK)], lsem)
            for b in range(NBUF):
                io = pl.multiple_of(p + b * CHUNK, 8)
                pltpu.make_async_copy(
                    xs_hbm.at[sidx_v.at[pl.ds(io, CHUNK)]],
                    rows[b], gsem[b]).wait()
                pltpu.async_copy(rows[b],
                                 P_sp.at[didx_v.at[pl.ds(io, CHUNK)]],
                                 ssem[b], add=True)
            pltpu.make_async_copy(src_hbm.at[pl.ds(nxt, BLK)],
                                  sidx_v.at[pl.ds(pn, BLK)], lsem).wait()
            pltpu.make_async_copy(dst_hbm.at[pl.ds(nxt, BLK)],
                                  didx_v.at[pl.ds(pn, BLK)], lsem).wait()
            for b in range(NBUF):
                io = pl.multiple_of(p + b * CHUNK, 8)
                ion = pl.multiple_of(pn + b * CHUNK, 8)
                pltpu.make_async_copy(
                    rows[b], P_sp.at[didx_v.at[pl.ds(io, CHUNK)]],
                    ssem[b]).wait()
                pltpu.async_copy(xs_hbm.at[sidx_v.at[pl.ds(ion, CHUNK)]],
                                 rows[b], gsem[b])
            return 0
        lax.fori_loop(0, NGROUPS - 1, grp, 0)

        lp = ((NGROUPS - 1) % 2) * BLK
        for b in range(NBUF):
            o = lp + b * CHUNK
            pltpu.make_async_copy(xs_hbm.at[sidx_v.at[pl.ds(o, CHUNK)]],
                                  rows[b], gsem[b]).wait()
            pltpu.async_copy(rows[b], P_sp.at[didx_v.at[pl.ds(o, CHUNK)]],
                             ssem[b], add=True)
        for b in range(NBUF):
            o = lp + b * CHUNK
            pltpu.make_async_copy(rows[b],
                                  P_sp.at[didx_v.at[pl.ds(o, CHUNK)]],
                                  ssem[b]).wait()

    @pl.when(c == 0)
    def _():
        edges(row_hbm, col_hbm, xso_hbm)

    @pl.when(c == 1)
    def _():
        edges(col_hbm, row_hbm, xsi_hbm)

    plsc.subcore_barrier()

    @pl.when(c == 0)
    def _():
        pltpu.sync_copy(P_sp.at[pl.ds(base_n, NODES_PER_SUB), :],
                        po_hbm.at[pl.ds(base_n, NODES_PER_SUB), :])

    @pl.when(c == 1)
    def _():
        pltpu.sync_copy(P_sp.at[pl.ds(base_n, NODES_PER_SUB), :],
                        pi_hbm.at[pl.ds(base_n, NODES_PER_SUB), :])


def _sc_prop(xs_o, xs_i, row, col):
    mesh = plsc.VectorSubcoreMesh(core_axis_name="c", subcore_axis_name="s")
    f32 = jnp.float32
    kern = functools.partial(
        pl.kernel,
        mesh=mesh,
        out_type=[
            jax.ShapeDtypeStruct((NPAD, F), f32),
            jax.ShapeDtypeStruct((NPAD, F), f32),
        ],
        scratch_types=[
            pltpu.VMEM_SHARED((NPAD, F), f32),
            pltpu.VMEM((2 * BLK,), jnp.int32),
            pltpu.VMEM((2 * BLK,), jnp.int32),
            pltpu.VMEM((CHUNK, F), f32),
            pltpu.VMEM((CHUNK, F), f32),
            pltpu.VMEM((CHUNK, F), f32),
            pltpu.VMEM((CHUNK, F), f32),
            pltpu.SemaphoreType.DMA,
            pltpu.SemaphoreType.DMA,
            pltpu.SemaphoreType.DMA,
            pltpu.SemaphoreType.DMA,
            pltpu.SemaphoreType.DMA,
            pltpu.SemaphoreType.DMA,
            pltpu.SemaphoreType.DMA,
            pltpu.SemaphoreType.DMA,
            pltpu.SemaphoreType.DMA,
        ],
    )(_sc_prop_body)
    return kern(xs_o, xs_i, row, col)


BN = 2000


def _tc_scale_body(x_ref, dego_ref, degi_ref, xso_ref, xsi_ref):
    x = x_ref[...]
    do = dego_ref[...]
    di = degi_ref[...]
    inv_o = jnp.where(do > 0.0, 1.0 / jnp.where(do > 0.0, do, 1.0), 0.0)
    inv_i = jnp.where(di > 0.0, 1.0 / jnp.where(di > 0.0, di, 1.0), 0.0)
    xso_ref[...] = x * inv_o
    xsi_ref[...] = x * inv_i


def _tc_scale(x, deg_o, deg_i):
    grid = (N // BN,)
    row_block = pl.BlockSpec((BN, F), lambda i: (i, 0))
    deg_block = pl.BlockSpec((BN, 1), lambda i: (i, 0))
    return pl.pallas_call(
        _tc_scale_body,
        grid=grid,
        in_specs=[row_block, deg_block, deg_block],
        out_specs=[row_block, row_block],
        out_shape=[
            jax.ShapeDtypeStruct((N, F), jnp.float32),
            jax.ShapeDtypeStruct((N, F), jnp.float32),
        ],
    )(x, deg_o, deg_i)


def _tc_tail_body(x_ref, po_ref, pi_ref, wz_ref, wh_ref, bz_ref, bh_ref,
                  wl_ref, bl_ref, o_ref):
    cat = jnp.concatenate([x_ref[...], po_ref[...], pi_ref[...]], axis=1)
    z = jax.nn.sigmoid(
        jnp.dot(cat, wz_ref[...], preferred_element_type=jnp.float32)
        + bz_ref[...])
    ht = jnp.tanh(
        jnp.dot(cat, wh_ref[...], preferred_element_type=jnp.float32)
        + bh_ref[...])
    h = (1.0 - z) * ht
    o_ref[...] = (
        jnp.dot(jnp.maximum(h, 0.0), wl_ref[...],
                preferred_element_type=jnp.float32)
        + bl_ref[...])


def _tc_tail(x, p_o, p_i, wz, wh, bz, bh, wl, bl):
    grid = (N // BN,)

    def full(i):
        return (0, 0)

    return pl.pallas_call(
        _tc_tail_body,
        grid=grid,
        in_specs=[
            pl.BlockSpec((BN, F), lambda i: (i, 0)),
            pl.BlockSpec((BN, F), lambda i: (i, 0)),
            pl.BlockSpec((BN, F), lambda i: (i, 0)),
            pl.BlockSpec((3 * F, HID), full),
            pl.BlockSpec((3 * F, HID), full),
            pl.BlockSpec((1, HID), full),
            pl.BlockSpec((1, HID), full),
            pl.BlockSpec((HID, 1), full),
            pl.BlockSpec((1, 1), full),
        ],
        out_specs=pl.BlockSpec((BN, 1), lambda i: (i, 0)),
        out_shape=jax.ShapeDtypeStruct((N, 1), jnp.float32),
    )(x, p_o, p_i, wz, wh, bz, bh, wl, bl)


def kernel(x, edge_index, edge_weight, W_z, b_z, W_r, b_r, W_h, b_h,
           W_lin, b_lin):
    del W_r, b_r
    ei = edge_index.astype(jnp.int32)
    row, col = ei[0], ei[1]

    deg_o, deg_i = _sc_deg(row, col, edge_weight)
    xs_o, xs_i = _tc_scale(
        x, deg_o[:N].reshape(N, 1), deg_i[:N].reshape(N, 1))
    p_o, p_i = _sc_prop(xs_o, xs_i, row, col)

    def cat_w(W):
        return jnp.concatenate([W[0, 0, :F] + W[1, 0, :F],
                                W[0, 1, :F], W[1, 1, :F]], axis=0)

    out = _tc_tail(
        x, p_o, p_i, cat_w(W_z), cat_w(W_h),
        b_z.reshape(1, HID), b_h.reshape(1, HID),
        W_lin, b_lin.reshape(1, 1))
    return out
